# baseline (speedup 1.0000x reference)
import functools

import jax
import jax.numpy as jnp
from jax import lax
from jax.experimental import pallas as pl
from jax.experimental.pallas import tpu as pltpu
from jax.experimental.pallas import tpu_sc as plsc

N = 524288
M = 16384
NC = 2
NS = 16
L = 16
NW = NC * NS
K = N // NW
SLC = M // NS
TS = 4096
SCHED = (1024, 1024, 2048, 4096, 4096, 4096)
OFF = (0, 1024, 2048, 4096, 8192, 12288)
NT = len(SCHED)
LCAP = TS + 256
Q = K // 4

_mesh = plsc.VectorSubcoreMesh(core_axis_name="c", subcore_axis_name="s")
_params = pltpu.CompilerParams(needs_layout_passes=False)


@functools.partial(
    pl.kernel,
    out_type=(jax.ShapeDtypeStruct((2 * M,), jnp.float32),
              jax.ShapeDtypeStruct((2 * M,), jnp.float32)),
    mesh=_mesh,
    compiler_params=_params,
    scratch_types=[
        pltpu.VMEM((TS,), jnp.float32),
        pltpu.VMEM((TS,), jnp.float32),
        pltpu.VMEM((TS,), jnp.float32),
        pltpu.VMEM((TS,), jnp.float32),
        pltpu.VMEM((TS,), jnp.int32),
        pltpu.VMEM((TS,), jnp.int32),
        pltpu.VMEM((TS + L,), jnp.int32),
        pltpu.VMEM((TS + L,), jnp.int32),
        pltpu.VMEM((SLC,), jnp.float32),
        pltpu.VMEM((LCAP,), jnp.int32),
        pltpu.VMEM((LCAP,), jnp.float32),
        pltpu.VMEM((LCAP,), jnp.float32),
        pltpu.VMEM((LCAP,), jnp.int32),
        pltpu.VMEM((LCAP,), jnp.float32),
        pltpu.VMEM((LCAP,), jnp.float32),
        pltpu.VMEM_SHARED((M,), jnp.float32),
        pltpu.VMEM_SHARED((M,), jnp.float32),
        pltpu.SemaphoreType.DMA,
        pltpu.SemaphoreType.DMA,
        pltpu.SemaphoreType.DMA,
    ],
)
def _partial_sums(en_hbm, hd_hbm, fc_hbm, seg_hbm, npart_hbm, dpart_hbm,
                  en_a, en_b, hd_a, hd_b, fc_a, fc_b, seg_a, seg_b, zb_v,
                  pi_v, pn_v, pd_v, mi_v, mn_v, md_v,
                  nacc_sh, dacc_sh, sem_st, sem_sc, sem_z):
    c = lax.axis_index("c")
    s = lax.axis_index("s")
    w = s * NC + c
    base = w * K

    ens = (en_a, en_b)
    hds = (hd_a, hd_b)
    fcs = (fc_a, fc_b)
    segs = (seg_a, seg_b)

    zi16 = jnp.zeros((L,), jnp.int32)
    zf16 = jnp.zeros((L,), jnp.float32)
    l15 = lax.iota(jnp.int32, L) == (L - 1)

    def stage(t):
        tb = base + OFF[t]
        sz = SCHED[t]
        cps = [pltpu.async_copy(en_hbm.at[pl.ds(tb, sz)],
                                ens[t % 2].at[pl.ds(0, sz)], sem_st),
               pltpu.async_copy(hd_hbm.at[pl.ds(tb, sz)],
                                hds[t % 2].at[pl.ds(0, sz)], sem_st),
               pltpu.async_copy(fc_hbm.at[pl.ds(tb, sz)],
                                fcs[t % 2].at[pl.ds(0, sz)], sem_st)]
        if t < NT - 1:
            cps.append(pltpu.async_copy(seg_hbm.at[pl.ds(tb, sz + L)],
                                        segs[t % 2].at[pl.ds(0, sz + L)],
                                        sem_st))
        else:
            @pl.when(w < NW - 1)
            def _():
                pltpu.sync_copy(seg_hbm.at[pl.ds(tb, sz + L)],
                                segs[t % 2].at[pl.ds(0, sz + L)])

            @pl.when(w == NW - 1)
            def _():
                pltpu.sync_copy(seg_hbm.at[pl.ds(tb, sz)],
                                segs[t % 2].at[pl.ds(0, sz)])
        return cps

    @plsc.parallel_loop(0, SLC, L, unroll=8)
    def zinit(i):
        zb_v[pl.ds(i, L)] = zf16

    zcps = [pltpu.async_copy(zb_v, nacc_sh.at[pl.ds(s * SLC, SLC)], sem_z),
            pltpu.async_copy(zb_v, dacc_sh.at[pl.ds(s * SLC, SLC)], sem_z)]
    pending = {0: stage(0), 1: stage(1)}
    for cp in zcps:
        cp.wait()
    plsc.subcore_barrier()

    for t in range(NT):
        for cp in pending.pop(t):
            cp.wait()
        en_v, hd_v, fc_v, seg_v = ens[t % 2], hds[t % 2], fcs[t % 2], segs[t % 2]

        @plsc.parallel_loop(0, SCHED[t], L, unroll=4,
                            carry=(jnp.int32(0), jnp.int32(0)))
        def grp(i, ptrs):
            pp, mp = ptrs
            sg = seg_v[pl.ds(i, L)]
            sgn = seg_v[pl.ds(i + 1, L)]
            en = en_v[pl.ds(i, L)]
            hd = hd_v[pl.ds(i, L)]
            inv = 1.0 / hd
            fcf = fc_v[pl.ds(i, L)].astype(jnp.float32)
            cs_n = plsc.cumsum(en * inv + fcf)
            cs_d = plsc.cumsum(inv)
            mb = sg != sgn
            plus_m = mb | l15
            minus_m = mb & jnp.logical_not(l15)
            plsc.store_compressed(pi_v.at[pl.ds(pp, L)], sg, mask=plus_m)
            plsc.store_compressed(pn_v.at[pl.ds(pp, L)], cs_n, mask=plus_m)
            plsc.store_compressed(pd_v.at[pl.ds(pp, L)], cs_d, mask=plus_m)
            plsc.store_compressed(mi_v.at[pl.ds(mp, L)], sgn, mask=minus_m)
            plsc.store_compressed(mn_v.at[pl.ds(mp, L)], -cs_n, mask=minus_m)
            plsc.store_compressed(md_v.at[pl.ds(mp, L)], -cs_d, mask=minus_m)
            pc = jnp.sum(plus_m.astype(jnp.int32))
            mc = jnp.sum(minus_m.astype(jnp.int32))
            return (pp + pc, mp + mc)

        pp, mp = grp

        if t + 2 < NT:
            pending[t + 2] = stage(t + 2)

        for q in range(8):
            pi_v[pl.ds(pp + q * L, L)] = zi16
            pn_v[pl.ds(pp + q * L, L)] = zf16
            pd_v[pl.ds(pp + q * L, L)] = zf16
            mi_v[pl.ds(mp + q * L, L)] = zi16
            mn_v[pl.ds(mp + q * L, L)] = zf16
            md_v[pl.ds(mp + q * L, L)] = zf16

        def make_blk(idx_list, nval_list, dval_list):
            def blk(j, _):
                idx = idx_list.at[pl.ds(j * 128, 128)]
                cp1 = pltpu.async_copy(nval_list.at[pl.ds(j * 128, 128)],
                                       nacc_sh.at[idx], sem_sc, add=True)
                cp2 = pltpu.async_copy(dval_list.at[pl.ds(j * 128, 128)],
                                       dacc_sh.at[idx], sem_sc, add=True)
                cp1.wait()
                cp2.wait()
                return 0
            return blk

        lax.fori_loop(0, (pp + 127) // 128, make_blk(pi_v, pn_v, pd_v), 0)
        lax.fori_loop(0, (mp + 127) // 128, make_blk(mi_v, mn_v, md_v), 0)

    plsc.subcore_barrier()

    d1 = pltpu.async_copy(nacc_sh.at[pl.ds(s * SLC, SLC)],
                          npart_hbm.at[pl.ds(c * M + s * SLC, SLC)], sem_z)
    d2 = pltpu.async_copy(dacc_sh.at[pl.ds(s * SLC, SLC)],
                          dpart_hbm.at[pl.ds(c * M + s * SLC, SLC)], sem_z)
    d1.wait()
    d2.wait()


@functools.partial(
    pl.kernel,
    out_type=jax.ShapeDtypeStruct((N,), jnp.float32),
    mesh=_mesh,
    compiler_params=_params,
    scratch_types=[
        pltpu.VMEM((M,), jnp.float32),
        pltpu.VMEM((K,), jnp.float32),
        pltpu.VMEM((K,), jnp.float32),
        pltpu.VMEM((K,), jnp.int32),
        pltpu.VMEM((Q,), jnp.float32),
        pltpu.VMEM((Q,), jnp.float32),
        pltpu.SemaphoreType.DMA,
        pltpu.SemaphoreType.DMA,
    ],
)
def _charges(en_hbm, hd_hbm, seg_hbm, ratio_hbm, out_hbm,
             ratio_v, en_v, hd_v, seg_v, out_a, out_b, sem_st, sem_out):
    c = lax.axis_index("c")
    s = lax.axis_index("s")
    w = s * NC + c
    base = w * K
    outs = (out_a, out_b)

    cps = [pltpu.async_copy(ratio_hbm, ratio_v, sem_st),
           pltpu.async_copy(en_hbm.at[pl.ds(base, K)], en_v, sem_st),
           pltpu.async_copy(hd_hbm.at[pl.ds(base, K)], hd_v, sem_st),
           pltpu.async_copy(seg_hbm.at[pl.ds(base, K)], seg_v, sem_st)]
    for cp in cps:
        cp.wait()

    ocps = {}
    for q in range(4):
        qb = q * Q
        out_v = outs[q % 2]
        if q >= 2:
            ocps.pop(q - 2).wait()

        @plsc.parallel_loop(0, Q, L, unroll=8)
        def abody(i):
            gi = qb + i
            en = en_v[pl.ds(gi, L)]
            hd = hd_v[pl.ds(gi, L)]
            inv = 1.0 / hd
            sg = seg_v[pl.ds(gi, L)]
            rt = plsc.load_gather(ratio_v, [sg])
            out_v[pl.ds(i, L)] = inv * rt - en * inv

        ocps[q] = pltpu.async_copy(out_v, out_hbm.at[pl.ds(base + qb, Q)],
                                   sem_out)
    ocps.pop(2).wait()
    ocps.pop(3).wait()


def kernel(inputs, formal_charge, segment_ids):
    en = inputs[:, 0]
    hd = inputs[:, 1]
    npart, dpart = _partial_sums(en, hd, formal_charge, segment_ids)
    ratio = (npart[:M] + npart[M:]) / (dpart[:M] + dpart[M:])
    out = _charges(en, hd, segment_ids, ratio)
    return out.reshape(N, 1)


# speedup vs baseline: 1.2244x; 1.2244x over previous
import functools

import jax
import jax.numpy as jnp
from jax import lax
from jax.experimental import pallas as pl
from jax.experimental.pallas import tpu as pltpu
from jax.experimental.pallas import tpu_sc as plsc

N = 524288
M = 16384
NC = 2
NS = 16
L = 16
NW = NC * NS
K = N // NW
SLC = M // NS
TS = 8192
NT = K // TS
LCAP = TS + 256
Q = K // 4

_mesh = plsc.VectorSubcoreMesh(core_axis_name="c", subcore_axis_name="s")
_params = pltpu.CompilerParams(needs_layout_passes=False)


@functools.partial(
    pl.kernel,
    out_type=(jax.ShapeDtypeStruct((2 * M,), jnp.float32),
              jax.ShapeDtypeStruct((2 * M,), jnp.float32)),
    mesh=_mesh,
    compiler_params=_params,
    scratch_types=[
        pltpu.VMEM((TS,), jnp.float32),
        pltpu.VMEM((TS,), jnp.float32),
        pltpu.VMEM((TS,), jnp.float32),
        pltpu.VMEM((TS,), jnp.float32),
        pltpu.VMEM((TS,), jnp.int32),
        pltpu.VMEM((TS,), jnp.int32),
        pltpu.VMEM((TS + L,), jnp.int32),
        pltpu.VMEM((TS + L,), jnp.int32),
        pltpu.VMEM((SLC,), jnp.float32),
        pltpu.VMEM((LCAP,), jnp.int32),
        pltpu.VMEM((LCAP,), jnp.float32),
        pltpu.VMEM((LCAP,), jnp.float32),
        pltpu.VMEM((LCAP,), jnp.int32),
        pltpu.VMEM((LCAP,), jnp.float32),
        pltpu.VMEM((LCAP,), jnp.float32),
        pltpu.VMEM_SHARED((M,), jnp.float32),
        pltpu.VMEM_SHARED((M,), jnp.float32),
        pltpu.SemaphoreType.DMA,
        pltpu.SemaphoreType.DMA,
        pltpu.SemaphoreType.DMA,
    ],
)
def _partial_sums(en_hbm, hd_hbm, fc_hbm, seg_hbm, npart_hbm, dpart_hbm,
                  en_a, en_b, hd_a, hd_b, fc_a, fc_b, seg_a, seg_b, zb_v,
                  pi_v, pn_v, pd_v, mi_v, mn_v, md_v,
                  nacc_sh, dacc_sh, sem_st, sem_sc, sem_z):
    c = lax.axis_index("c")
    s = lax.axis_index("s")
    w = s * NC + c
    base = w * K

    ens = (en_a, en_b)
    hds = (hd_a, hd_b)
    fcs = (fc_a, fc_b)
    segs = (seg_a, seg_b)

    zi16 = jnp.zeros((L,), jnp.int32)
    zf16 = jnp.zeros((L,), jnp.float32)
    l15 = lax.iota(jnp.int32, L) == (L - 1)

    def stage(t):
        tb = base + t * TS
        cps = [pltpu.async_copy(en_hbm.at[pl.ds(tb, TS)], ens[t % 2], sem_st),
               pltpu.async_copy(hd_hbm.at[pl.ds(tb, TS)], hds[t % 2], sem_st),
               pltpu.async_copy(fc_hbm.at[pl.ds(tb, TS)], fcs[t % 2], sem_st)]
        if t < NT - 1:
            cps.append(pltpu.async_copy(seg_hbm.at[pl.ds(tb, TS + L)],
                                        segs[t % 2], sem_st))
        else:
            @pl.when(w < NW - 1)
            def _():
                pltpu.sync_copy(seg_hbm.at[pl.ds(tb, TS + L)], segs[t % 2])

            @pl.when(w == NW - 1)
            def _():
                pltpu.sync_copy(seg_hbm.at[pl.ds(tb, TS)],
                                segs[t % 2].at[pl.ds(0, TS)])
        return cps

    @plsc.parallel_loop(0, SLC, L, unroll=8)
    def zinit(i):
        zb_v[pl.ds(i, L)] = zf16

    zcps = [pltpu.async_copy(zb_v, nacc_sh.at[pl.ds(s * SLC, SLC)], sem_z),
            pltpu.async_copy(zb_v, dacc_sh.at[pl.ds(s * SLC, SLC)], sem_z)]
    pending = {0: stage(0), 1: stage(1)}
    for cp in zcps:
        cp.wait()
    plsc.subcore_barrier()

    for t in range(NT):
        for cp in pending.pop(t):
            cp.wait()
        en_v, hd_v, fc_v, seg_v = ens[t % 2], hds[t % 2], fcs[t % 2], segs[t % 2]

        @plsc.parallel_loop(0, TS, L, unroll=4,
                            carry=(jnp.int32(0), jnp.int32(0)))
        def grp(i, ptrs):
            pp, mp = ptrs
            sg = seg_v[pl.ds(i, L)]
            sgn = seg_v[pl.ds(i + 1, L)]
            en = en_v[pl.ds(i, L)]
            hd = hd_v[pl.ds(i, L)]
            inv = 1.0 / hd
            fcf = fc_v[pl.ds(i, L)].astype(jnp.float32)
            cs_n = plsc.cumsum(en * inv + fcf)
            cs_d = plsc.cumsum(inv)
            mb = sg != sgn
            plus_m = mb | l15
            minus_m = mb & jnp.logical_not(l15)
            plsc.store_compressed(pi_v.at[pl.ds(pp, L)], sg, mask=plus_m)
            plsc.store_compressed(pn_v.at[pl.ds(pp, L)], cs_n, mask=plus_m)
            plsc.store_compressed(pd_v.at[pl.ds(pp, L)], cs_d, mask=plus_m)
            plsc.store_compressed(mi_v.at[pl.ds(mp, L)], sgn, mask=minus_m)
            plsc.store_compressed(mn_v.at[pl.ds(mp, L)], -cs_n, mask=minus_m)
            plsc.store_compressed(md_v.at[pl.ds(mp, L)], -cs_d, mask=minus_m)
            pc = jnp.sum(plus_m.astype(jnp.int32))
            mc = jnp.sum(minus_m.astype(jnp.int32))
            return (pp + pc, mp + mc)

        pp, mp = grp

        if t + 2 < NT:
            pending[t + 2] = stage(t + 2)

        for q in range(8):
            pi_v[pl.ds(pp + q * L, L)] = zi16
            pn_v[pl.ds(pp + q * L, L)] = zf16
            pd_v[pl.ds(pp + q * L, L)] = zf16
            mi_v[pl.ds(mp + q * L, L)] = zi16
            mn_v[pl.ds(mp + q * L, L)] = zf16
            md_v[pl.ds(mp + q * L, L)] = zf16

        def make_blk(idx_list, nval_list, dval_list):
            def blk(j, _):
                idx = idx_list.at[pl.ds(j * 128, 128)]
                cp1 = pltpu.async_copy(nval_list.at[pl.ds(j * 128, 128)],
                                       nacc_sh.at[idx], sem_sc, add=True)
                cp2 = pltpu.async_copy(dval_list.at[pl.ds(j * 128, 128)],
                                       dacc_sh.at[idx], sem_sc, add=True)
                cp1.wait()
                cp2.wait()
                return 0
            return blk

        lax.fori_loop(0, (pp + 127) // 128, make_blk(pi_v, pn_v, pd_v), 0)
        lax.fori_loop(0, (mp + 127) // 128, make_blk(mi_v, mn_v, md_v), 0)

    plsc.subcore_barrier()

    d1 = pltpu.async_copy(nacc_sh.at[pl.ds(s * SLC, SLC)],
                          npart_hbm.at[pl.ds(c * M + s * SLC, SLC)], sem_z)
    d2 = pltpu.async_copy(dacc_sh.at[pl.ds(s * SLC, SLC)],
                          dpart_hbm.at[pl.ds(c * M + s * SLC, SLC)], sem_z)
    d1.wait()
    d2.wait()


@functools.partial(
    pl.kernel,
    out_type=jax.ShapeDtypeStruct((N,), jnp.float32),
    mesh=_mesh,
    compiler_params=_params,
    scratch_types=[
        pltpu.VMEM((M,), jnp.float32),
        pltpu.VMEM((K,), jnp.float32),
        pltpu.VMEM((K,), jnp.float32),
        pltpu.VMEM((K,), jnp.int32),
        pltpu.VMEM((Q,), jnp.float32),
        pltpu.VMEM((Q,), jnp.float32),
        pltpu.SemaphoreType.DMA,
        pltpu.SemaphoreType.DMA,
    ],
)
def _charges(en_hbm, hd_hbm, seg_hbm, ratio_hbm, out_hbm,
             ratio_v, en_v, hd_v, seg_v, out_a, out_b, sem_st, sem_out):
    c = lax.axis_index("c")
    s = lax.axis_index("s")
    w = s * NC + c
    base = w * K
    outs = (out_a, out_b)

    cps = [pltpu.async_copy(ratio_hbm, ratio_v, sem_st),
           pltpu.async_copy(en_hbm.at[pl.ds(base, K)], en_v, sem_st),
           pltpu.async_copy(hd_hbm.at[pl.ds(base, K)], hd_v, sem_st),
           pltpu.async_copy(seg_hbm.at[pl.ds(base, K)], seg_v, sem_st)]
    for cp in cps:
        cp.wait()

    ocps = {}
    for q in range(4):
        qb = q * Q
        out_v = outs[q % 2]
        if q >= 2:
            ocps.pop(q - 2).wait()

        @plsc.parallel_loop(0, Q, L, unroll=8)
        def abody(i):
            gi = qb + i
            en = en_v[pl.ds(gi, L)]
            hd = hd_v[pl.ds(gi, L)]
            inv = 1.0 / hd
            sg = seg_v[pl.ds(gi, L)]
            rt = plsc.load_gather(ratio_v, [sg])
            out_v[pl.ds(i, L)] = inv * rt - en * inv

        ocps[q] = pltpu.async_copy(out_v, out_hbm.at[pl.ds(base + qb, Q)],
                                   sem_out)
    ocps.pop(2).wait()
    ocps.pop(3).wait()


def kernel(inputs, formal_charge, segment_ids):
    en = inputs[:, 0]
    hd = inputs[:, 1]
    npart, dpart = _partial_sums(en, hd, formal_charge, segment_ids)
    ratio = (npart[:M] + npart[M:]) / (dpart[:M] + dpart[M:])
    out = _charges(en, hd, segment_ids, ratio)
    return out.reshape(N, 1)


# speedup vs baseline: 1.2332x; 1.0072x over previous
import functools

import jax
import jax.numpy as jnp
from jax import lax
from jax.experimental import pallas as pl
from jax.experimental.pallas import tpu as pltpu
from jax.experimental.pallas import tpu_sc as plsc

N = 524288
M = 16384
NC = 2
NS = 16
L = 16
NW = NC * NS
K = N // NW
SLC = M // NS
TS = 8192
NT = K // TS
LCAP = TS + 256
Q = K // 2

_mesh = plsc.VectorSubcoreMesh(core_axis_name="c", subcore_axis_name="s")
_params = pltpu.CompilerParams(needs_layout_passes=False)


@functools.partial(
    pl.kernel,
    out_type=(jax.ShapeDtypeStruct((2 * M,), jnp.float32),
              jax.ShapeDtypeStruct((2 * M,), jnp.float32)),
    mesh=_mesh,
    compiler_params=_params,
    scratch_types=[
        pltpu.VMEM((TS,), jnp.float32),
        pltpu.VMEM((TS,), jnp.float32),
        pltpu.VMEM((TS,), jnp.float32),
        pltpu.VMEM((TS,), jnp.float32),
        pltpu.VMEM((TS,), jnp.int32),
        pltpu.VMEM((TS,), jnp.int32),
        pltpu.VMEM((TS + L,), jnp.int32),
        pltpu.VMEM((TS + L,), jnp.int32),
        pltpu.VMEM((SLC,), jnp.float32),
        pltpu.VMEM((LCAP,), jnp.int32),
        pltpu.VMEM((LCAP,), jnp.float32),
        pltpu.VMEM((LCAP,), jnp.float32),
        pltpu.VMEM((LCAP,), jnp.int32),
        pltpu.VMEM((LCAP,), jnp.float32),
        pltpu.VMEM((LCAP,), jnp.float32),
        pltpu.VMEM_SHARED((M,), jnp.float32),
        pltpu.VMEM_SHARED((M,), jnp.float32),
        pltpu.SemaphoreType.DMA,
        pltpu.SemaphoreType.DMA,
        pltpu.SemaphoreType.DMA,
    ],
)
def _partial_sums(en_hbm, hd_hbm, fc_hbm, seg_hbm, npart_hbm, dpart_hbm,
                  en_a, en_b, hd_a, hd_b, fc_a, fc_b, seg_a, seg_b, zb_v,
                  pi_v, pn_v, pd_v, mi_v, mn_v, md_v,
                  nacc_sh, dacc_sh, sem_st, sem_sc, sem_z):
    c = lax.axis_index("c")
    s = lax.axis_index("s")
    w = s * NC + c
    base = w * K

    ens = (en_a, en_b)
    hds = (hd_a, hd_b)
    fcs = (fc_a, fc_b)
    segs = (seg_a, seg_b)

    zi16 = jnp.zeros((L,), jnp.int32)
    zf16 = jnp.zeros((L,), jnp.float32)
    l15 = lax.iota(jnp.int32, L) == (L - 1)

    def stage(t):
        tb = base + t * TS
        cps = [pltpu.async_copy(en_hbm.at[pl.ds(tb, TS)], ens[t % 2], sem_st),
               pltpu.async_copy(hd_hbm.at[pl.ds(tb, TS)], hds[t % 2], sem_st),
               pltpu.async_copy(fc_hbm.at[pl.ds(tb, TS)], fcs[t % 2], sem_st)]
        if t < NT - 1:
            cps.append(pltpu.async_copy(seg_hbm.at[pl.ds(tb, TS + L)],
                                        segs[t % 2], sem_st))
        else:
            @pl.when(w < NW - 1)
            def _():
                pltpu.sync_copy(seg_hbm.at[pl.ds(tb, TS + L)], segs[t % 2])

            @pl.when(w == NW - 1)
            def _():
                pltpu.sync_copy(seg_hbm.at[pl.ds(tb, TS)],
                                segs[t % 2].at[pl.ds(0, TS)])
        return cps

    @plsc.parallel_loop(0, SLC, L, unroll=8)
    def zinit(i):
        zb_v[pl.ds(i, L)] = zf16

    zcps = [pltpu.async_copy(zb_v, nacc_sh.at[pl.ds(s * SLC, SLC)], sem_z),
            pltpu.async_copy(zb_v, dacc_sh.at[pl.ds(s * SLC, SLC)], sem_z)]
    pending = {0: stage(0), 1: stage(1)}
    for cp in zcps:
        cp.wait()
    plsc.subcore_barrier()

    for t in range(NT):
        for cp in pending.pop(t):
            cp.wait()
        en_v, hd_v, fc_v, seg_v = ens[t % 2], hds[t % 2], fcs[t % 2], segs[t % 2]

        @plsc.parallel_loop(0, TS, L, unroll=4,
                            carry=(jnp.int32(0), jnp.int32(0)))
        def grp(i, ptrs):
            pp, mp = ptrs
            sg = seg_v[pl.ds(i, L)]
            sgn = seg_v[pl.ds(i + 1, L)]
            en = en_v[pl.ds(i, L)]
            hd = hd_v[pl.ds(i, L)]
            inv = 1.0 / hd
            fcf = fc_v[pl.ds(i, L)].astype(jnp.float32)
            cs_n = plsc.cumsum(en * inv + fcf)
            cs_d = plsc.cumsum(inv)
            mb = sg != sgn
            plus_m = mb | l15
            minus_m = mb & jnp.logical_not(l15)
            plsc.store_compressed(pi_v.at[pl.ds(pp, L)], sg, mask=plus_m)
            plsc.store_compressed(pn_v.at[pl.ds(pp, L)], cs_n, mask=plus_m)
            plsc.store_compressed(pd_v.at[pl.ds(pp, L)], cs_d, mask=plus_m)
            plsc.store_compressed(mi_v.at[pl.ds(mp, L)], sgn, mask=minus_m)
            plsc.store_compressed(mn_v.at[pl.ds(mp, L)], -cs_n, mask=minus_m)
            plsc.store_compressed(md_v.at[pl.ds(mp, L)], -cs_d, mask=minus_m)
            pc = jnp.sum(plus_m.astype(jnp.int32))
            mc = jnp.sum(minus_m.astype(jnp.int32))
            return (pp + pc, mp + mc)

        pp, mp = grp

        if t + 2 < NT:
            pending[t + 2] = stage(t + 2)

        for q in range(8):
            pi_v[pl.ds(pp + q * L, L)] = zi16
            pn_v[pl.ds(pp + q * L, L)] = zf16
            pd_v[pl.ds(pp + q * L, L)] = zf16
            mi_v[pl.ds(mp + q * L, L)] = zi16
            mn_v[pl.ds(mp + q * L, L)] = zf16
            md_v[pl.ds(mp + q * L, L)] = zf16

        def make_blk(idx_list, nval_list, dval_list):
            def blk(j, _):
                idx = idx_list.at[pl.ds(j * 128, 128)]
                cp1 = pltpu.async_copy(nval_list.at[pl.ds(j * 128, 128)],
                                       nacc_sh.at[idx], sem_sc, add=True)
                cp2 = pltpu.async_copy(dval_list.at[pl.ds(j * 128, 128)],
                                       dacc_sh.at[idx], sem_sc, add=True)
                cp1.wait()
                cp2.wait()
                return 0
            return blk

        lax.fori_loop(0, (pp + 127) // 128, make_blk(pi_v, pn_v, pd_v), 0)
        lax.fori_loop(0, (mp + 127) // 128, make_blk(mi_v, mn_v, md_v), 0)

    plsc.subcore_barrier()

    d1 = pltpu.async_copy(nacc_sh.at[pl.ds(s * SLC, SLC)],
                          npart_hbm.at[pl.ds(c * M + s * SLC, SLC)], sem_z)
    d2 = pltpu.async_copy(dacc_sh.at[pl.ds(s * SLC, SLC)],
                          dpart_hbm.at[pl.ds(c * M + s * SLC, SLC)], sem_z)
    d1.wait()
    d2.wait()


@functools.partial(
    pl.kernel,
    out_type=jax.ShapeDtypeStruct((N,), jnp.float32),
    mesh=_mesh,
    compiler_params=_params,
    scratch_types=[
        pltpu.VMEM((M,), jnp.float32),
        pltpu.VMEM((K,), jnp.float32),
        pltpu.VMEM((K,), jnp.float32),
        pltpu.VMEM((K,), jnp.int32),
        pltpu.VMEM((Q,), jnp.float32),
        pltpu.VMEM((Q,), jnp.float32),
        pltpu.SemaphoreType.DMA,
        pltpu.SemaphoreType.DMA,
    ],
)
def _charges(en_hbm, hd_hbm, seg_hbm, ratio_hbm, out_hbm,
             ratio_v, en_v, hd_v, seg_v, out_a, out_b, sem_st, sem_out):
    c = lax.axis_index("c")
    s = lax.axis_index("s")
    w = s * NC + c
    base = w * K
    outs = (out_a, out_b)

    cps = [pltpu.async_copy(ratio_hbm, ratio_v, sem_st),
           pltpu.async_copy(en_hbm.at[pl.ds(base, K)], en_v, sem_st),
           pltpu.async_copy(hd_hbm.at[pl.ds(base, K)], hd_v, sem_st),
           pltpu.async_copy(seg_hbm.at[pl.ds(base, K)], seg_v, sem_st)]
    for cp in cps:
        cp.wait()

    ocps = {}
    for q in range(2):
        qb = q * Q
        out_v = outs[q % 2]

        @plsc.parallel_loop(0, Q, L, unroll=8)
        def abody(i):
            gi = qb + i
            en = en_v[pl.ds(gi, L)]
            hd = hd_v[pl.ds(gi, L)]
            inv = 1.0 / hd
            sg = seg_v[pl.ds(gi, L)]
            rt = plsc.load_gather(ratio_v, [sg])
            out_v[pl.ds(i, L)] = inv * rt - en * inv

        ocps[q] = pltpu.async_copy(out_v, out_hbm.at[pl.ds(base + qb, Q)],
                                   sem_out)
    ocps.pop(0).wait()
    ocps.pop(1).wait()


def kernel(inputs, formal_charge, segment_ids):
    en = inputs[:, 0]
    hd = inputs[:, 1]
    npart, dpart = _partial_sums(en, hd, formal_charge, segment_ids)
    ratio = (npart[:M] + npart[M:]) / (dpart[:M] + dpart[M:])
    out = _charges(en, hd, segment_ids, ratio)
    return out.reshape(N, 1)
